# Initial kernel scaffold; baseline (speedup 1.0000x reference)
import jax
import jax.numpy as jnp
from jax.experimental import pallas as pl


def kernel(gate_inputs, raw_inputs, W_gate, keys):
    raise NotImplementedError("write your pallas kernel here")


# speedup vs baseline: 8.7878x; 8.7878x over previous
import functools

import jax
import jax.numpy as jnp
from jax.experimental import pallas as pl
from jax.experimental.pallas import tpu as pltpu

X_DIM = 768
KEY_DIM = 128
N_EXPERTS = 8
T_TOKENS = 8192
BT = 512
NEG = -1e30


def _router_body(gate_ref, raw_ref, w_ref, keysT_ref, comb_ref, scores_ref):
    q = jnp.dot(gate_ref[...], w_ref[...], preferred_element_type=jnp.float32)
    logits = jnp.dot(q, keysT_ref[...], preferred_element_type=jnp.float32)
    lane = jax.lax.broadcasted_iota(jnp.int32, logits.shape, 1)
    logits = jnp.where(lane < N_EXPERTS, logits, NEG)

    m1 = jnp.max(logits, axis=-1, keepdims=True)
    a1 = jnp.min(jnp.where(logits == m1, lane, KEY_DIM), axis=-1, keepdims=True)
    l2 = jnp.where(lane == a1, NEG, logits)
    m2 = jnp.max(l2, axis=-1, keepdims=True)
    a2 = jnp.min(jnp.where(l2 == m2, lane, KEY_DIM), axis=-1, keepdims=True)

    d = jnp.exp(m2 - m1)
    denom = 1.0 + d
    w1 = 1.0 / denom
    w2 = d / denom

    scores = jnp.where(lane == a1, w1, 0.0) + jnp.where(lane == a2, w2, 0.0)
    scores_ref[...] = scores[:, :N_EXPERTS]
    comb_ref[...] = raw_ref[...] * (w1 + w2)


@jax.jit
def kernel(gate_inputs, raw_inputs, W_gate, keys):
    keysT = jnp.zeros((KEY_DIM, KEY_DIM), jnp.float32).at[:, :N_EXPERTS].set(keys.T)
    grid = (T_TOKENS // BT,)
    comb, scores = pl.pallas_call(
        _router_body,
        grid=grid,
        in_specs=[
            pl.BlockSpec((BT, X_DIM), lambda i: (i, 0)),
            pl.BlockSpec((BT, X_DIM), lambda i: (i, 0)),
            pl.BlockSpec((X_DIM, KEY_DIM), lambda i: (0, 0)),
            pl.BlockSpec((KEY_DIM, KEY_DIM), lambda i: (0, 0)),
        ],
        out_specs=[
            pl.BlockSpec((BT, X_DIM), lambda i: (i, 0)),
            pl.BlockSpec((BT, N_EXPERTS), lambda i: (i, 0)),
        ],
        out_shape=[
            jax.ShapeDtypeStruct((T_TOKENS, X_DIM), jnp.float32),
            jax.ShapeDtypeStruct((T_TOKENS, N_EXPERTS), jnp.float32),
        ],
    )(gate_inputs, raw_inputs, W_gate, keysT)
    return (comb, scores)
